# baseline (speedup 1.0000x reference)
import functools

import jax
import jax.numpy as jnp
from jax import lax
from jax.experimental import pallas as pl
from jax.experimental.pallas import tpu as pltpu
from jax.experimental.pallas import tpu_sc as plsc


def kernel(sequence, pos_table):
    B, L = sequence.shape
    D = pos_table.shape[1]
    table = pos_table[:L]

    info = plsc.get_sparse_core_info()
    NC, NS = 1, info.num_subcores
    NW = NC * NS
    rows_per_w = B // NW
    REP = 4
    n_chunks = rows_per_w // REP

    mesh = plsc.VectorSubcoreMesh(
        core_axis_name="c", subcore_axis_name="s", num_cores=1
    )

    @functools.partial(
        pl.kernel,
        mesh=mesh,
        out_type=jax.ShapeDtypeStruct((B, L, D), jnp.float32),
        scratch_types=[
            pltpu.VMEM((REP, L, D), jnp.float32),
            pltpu.SemaphoreType.DMA,
        ],
    )
    def pe_kernel(table_hbm, out_hbm, rep_v, sem):
        wid = lax.axis_index("s") * NC + lax.axis_index("c")
        base = wid * rows_per_w
        for j in range(REP):
            pltpu.sync_copy(table_hbm, rep_v.at[j])
        copies = [
            pltpu.async_copy(rep_v, out_hbm.at[pl.ds(base + i * REP, REP)], sem)
            for i in range(n_chunks)
        ]
        for c in copies:
            c.wait()

    return pe_kernel(table)


# speedup vs baseline: 1.2158x; 1.2158x over previous
import functools

import jax
import jax.numpy as jnp
from jax import lax
from jax.experimental import pallas as pl
from jax.experimental.pallas import tpu as pltpu
from jax.experimental.pallas import tpu_sc as plsc


def kernel(sequence, pos_table):
    B, L = sequence.shape
    D = pos_table.shape[1]
    table = pos_table[:L]

    info = plsc.get_sparse_core_info()
    NC, NS = info.num_cores, info.num_subcores
    NW = NC * NS
    rows_per_w = B // NW
    REP = 4
    n_chunks = rows_per_w // REP

    mesh = plsc.VectorSubcoreMesh(core_axis_name="c", subcore_axis_name="s")

    @functools.partial(
        pl.kernel,
        mesh=mesh,
        out_type=jax.ShapeDtypeStruct((B, L, D), jnp.float32),
        scratch_types=[
            pltpu.VMEM((REP, L, D), jnp.float32),
            pltpu.SemaphoreType.DMA,
        ],
    )
    def pe_kernel(table_hbm, out_hbm, rep_v, sem):
        wid = lax.axis_index("s") * NC + lax.axis_index("c")
        base = wid * rows_per_w
        stages = [pltpu.async_copy(table_hbm, rep_v.at[j], sem) for j in range(REP)]
        for c in stages:
            c.wait()
        copies = [
            pltpu.async_copy(rep_v, out_hbm.at[pl.ds(base + i * REP, REP)], sem)
            for i in range(n_chunks)
        ]
        for c in copies:
            c.wait()

    return pe_kernel(table)
